# Initial kernel scaffold; baseline (speedup 1.0000x reference)
import jax
import jax.numpy as jnp
from jax.experimental import pallas as pl


def kernel(x, index, weight):
    raise NotImplementedError("write your pallas kernel here")


# speedup vs baseline: 128.6763x; 128.6763x over previous
import functools

import jax
import jax.numpy as jnp
from jax import lax
from jax.experimental import pallas as pl
from jax.experimental.pallas import tpu as pltpu
from jax.experimental.pallas import tpu_sc as plsc

R, C = 16384, 200
N = R * C
V = 1_000_000

NC, NS = 2, 16
NW = NC * NS
PER_W = N // NW
BLK = 6400
NBLK = PER_W // BLK
STAGERS = 8
STAGE_CHUNK = V // STAGERS
STAGE_PIECE = 5000
STAGE_PIECES = STAGE_CHUNK // STAGE_PIECE
LANES = 16


def _body(x_hbm, idx_hbm, w_hbm, out_hbm, idx_v, x_v, w_v, table, sem):
    cid = lax.axis_index("c")
    sid = lax.axis_index("s")
    wid = sid * NC + cid

    @pl.when(sid < STAGERS)
    def _():
        def piece(k, c):
            off = sid * STAGE_CHUNK + k * STAGE_PIECE
            pltpu.sync_copy(w_hbm.at[pl.ds(off, STAGE_PIECE)],
                            x_v.at[pl.ds(0, STAGE_PIECE)])
            pltpu.sync_copy(x_v.at[pl.ds(0, STAGE_PIECE)],
                            table.at[pl.ds(off, STAGE_PIECE)])
            return c

        lax.fori_loop(0, STAGE_PIECES, piece, 0)

    plsc.subcore_barrier()

    base = wid * PER_W

    def block(b, carry):
        off = base + b * BLK
        pltpu.sync_copy(idx_hbm.at[pl.ds(off, BLK)], idx_v)
        pltpu.sync_copy(x_hbm.at[pl.ds(off, BLK)], x_v)
        pltpu.async_copy(table.at[idx_v], w_v, sem).wait()

        def mul(i, c):
            sl = pl.ds(i * LANES, LANES)
            x_v[sl] = x_v[sl] * w_v[sl]
            return c

        lax.fori_loop(0, BLK // LANES, mul, 0, unroll=8)
        pltpu.sync_copy(x_v, out_hbm.at[pl.ds(off, BLK)])
        return carry

    lax.fori_loop(0, NBLK, block, 0)


@jax.jit
def kernel(x, index, weight):
    mesh = plsc.VectorSubcoreMesh(core_axis_name="c", subcore_axis_name="s")
    run = functools.partial(
        pl.kernel,
        mesh=mesh,
        out_type=jax.ShapeDtypeStruct((N,), jnp.float32),
        scratch_types=[
            pltpu.VMEM((BLK,), jnp.int32),
            pltpu.VMEM((BLK,), jnp.float32),
            pltpu.VMEM((BLK,), jnp.float32),
            pltpu.VMEM_SHARED((V,), jnp.float32),
            pltpu.SemaphoreType.DMA,
        ],
    )(_body)
    out = run(x.reshape(N), index.reshape(N).astype(jnp.int32), weight)
    return out.reshape(R, C)


# speedup vs baseline: 151.1971x; 1.1750x over previous
import functools

import jax
import jax.numpy as jnp
from jax import lax
from jax.experimental import pallas as pl
from jax.experimental.pallas import tpu as pltpu
from jax.experimental.pallas import tpu_sc as plsc

R, C = 16384, 200
N = R * C
V = 1_000_000

NC, NS = 2, 16
NW = NC * NS
PER_W = N // NW
BLK = 6400
NBLK = PER_W // BLK
STAGERS = 8
STAGE_CHUNK = V // STAGERS
STAGE_PIECE = 5000
STAGE_PIECES = STAGE_CHUNK // STAGE_PIECE
LANES = 16


def _body(x_hbm, idx_hbm, w_hbm, out_hbm,
          idx_v0, idx_v1, x_v0, x_v1, w_v0, w_v1, o_v0, o_v1, st_v, table,
          in_s0, in_s1, g_s0, g_s1, o_s0, o_s1):
    cid = lax.axis_index("c")
    sid = lax.axis_index("s")
    wid = sid * NC + cid
    base = wid * PER_W

    idx_b = (idx_v0, idx_v1)
    x_b = (x_v0, x_v1)
    w_b = (w_v0, w_v1)
    o_b = (o_v0, o_v1)
    in_s = (in_s0, in_s1)
    g_s = (g_s0, g_s1)
    o_s = (o_s0, o_s1)

    def in_copies(b):
        p = b % 2
        off = base + b * BLK
        return (pltpu.make_async_copy(idx_hbm.at[pl.ds(off, BLK)],
                                      idx_b[p], in_s[p]),
                pltpu.make_async_copy(x_hbm.at[pl.ds(off, BLK)],
                                     x_b[p], in_s[p]))

    def gather_copy(b):
        p = b % 2
        return pltpu.make_async_copy(table.at[idx_b[p]], w_b[p], g_s[p])

    def out_copy(b):
        p = b % 2
        off = base + b * BLK
        return pltpu.make_async_copy(o_b[p], out_hbm.at[pl.ds(off, BLK)],
                                     o_s[p])

    for c in in_copies(0):
        c.start()
    for c in in_copies(1):
        c.start()

    @pl.when(sid < STAGERS)
    def _():
        def piece(k, c):
            off = sid * STAGE_CHUNK + k * STAGE_PIECE
            pltpu.sync_copy(w_hbm.at[pl.ds(off, STAGE_PIECE)], st_v)
            pltpu.sync_copy(st_v, table.at[pl.ds(off, STAGE_PIECE)])
            return c

        lax.fori_loop(0, STAGE_PIECES, piece, 0)

    plsc.subcore_barrier()

    for c in in_copies(0):
        c.wait()
    gather_copy(0).start()

    def mul(b):
        p = b % 2

        def step(i, c):
            sl = pl.ds(i * LANES, LANES)
            o_b[p][sl] = x_b[p][sl] * w_b[p][sl]
            return c

        lax.fori_loop(0, BLK // LANES, step, 0, unroll=8)

    for b in range(NBLK):
        if b + 1 < NBLK:
            for c in in_copies(b + 1):
                c.wait()
            gather_copy(b + 1).start()
        gather_copy(b).wait()
        if b >= 2:
            out_copy(b - 2).wait()
        mul(b)
        out_copy(b).start()
        if b + 2 < NBLK:
            for c in in_copies(b + 2):
                c.start()

    out_copy(NBLK - 2).wait()
    out_copy(NBLK - 1).wait()


@jax.jit
def kernel(x, index, weight):
    mesh = plsc.VectorSubcoreMesh(core_axis_name="c", subcore_axis_name="s")
    run = functools.partial(
        pl.kernel,
        mesh=mesh,
        out_type=jax.ShapeDtypeStruct((N,), jnp.float32),
        scratch_types=[
            pltpu.VMEM((BLK,), jnp.int32),
            pltpu.VMEM((BLK,), jnp.int32),
            pltpu.VMEM((BLK,), jnp.float32),
            pltpu.VMEM((BLK,), jnp.float32),
            pltpu.VMEM((BLK,), jnp.float32),
            pltpu.VMEM((BLK,), jnp.float32),
            pltpu.VMEM((BLK,), jnp.float32),
            pltpu.VMEM((BLK,), jnp.float32),
            pltpu.VMEM((STAGE_PIECE,), jnp.float32),
            pltpu.VMEM_SHARED((V,), jnp.float32),
            pltpu.SemaphoreType.DMA,
            pltpu.SemaphoreType.DMA,
            pltpu.SemaphoreType.DMA,
            pltpu.SemaphoreType.DMA,
            pltpu.SemaphoreType.DMA,
            pltpu.SemaphoreType.DMA,
        ],
    )(_body)
    out = run(x.reshape(N), index.reshape(N).astype(jnp.int32), weight)
    return out.reshape(R, C)
